# baseline (speedup 1.0000x reference)
import functools

import jax
import jax.numpy as jnp
from jax import lax
from jax.experimental import pallas as pl
from jax.experimental.pallas import tpu as pltpu
from jax.experimental.pallas import tpu_sc as plsc

LANES = 16


def _dechunk_sc(zf, pf, bf, B, T, L, D):
    info = plsc.get_sparse_core_info()
    NC, NS = info.num_cores, info.num_subcores
    NW = NC * NS
    CPB = NW // B
    TCH = T // CPB
    S = 32
    NSUB = TCH // S
    NVC = TCH // LANES
    UD = D // LANES

    mesh = plsc.VectorSubcoreMesh(core_axis_name="c", subcore_axis_name="s")

    @functools.partial(
        pl.kernel,
        out_type=jax.ShapeDtypeStruct((B * T, D), jnp.float32),
        mesh=mesh,
        scratch_types=[
            pltpu.VMEM((T + LANES,), jnp.int32),
            pltpu.VMEM((TCH + LANES,), jnp.float32),
            pltpu.VMEM((TCH + 2 * LANES,), jnp.int32),
            pltpu.VMEM((T + 2 * LANES,), jnp.int32),
            pltpu.VMEM((LANES,), jnp.int32),
            pltpu.VMEM((16, LANES), jnp.int32),
            pltpu.VMEM_SHARED((16, LANES), jnp.int32),
            pltpu.VMEM((8, D), jnp.float32),
            pltpu.VMEM((S, D), jnp.float32),
            pltpu.VMEM((S, D), jnp.float32),
            pltpu.VMEM((S, D), jnp.float32),
            pltpu.VMEM((S, D), jnp.float32),
            pltpu.VMEM((S, D), jnp.float32),
            pltpu.SemaphoreType.DMA,
            pltpu.SemaphoreType.DMA,
            pltpu.SemaphoreType.DMA,
            pltpu.SemaphoreType.DMA,
            pltpu.SemaphoreType.DMA,
            pltpu.SemaphoreType.DMA,
        ],
    )
    def body(z_hbm, p_hbm, b_hbm, out_hbm, b_v, p_v, c_v, idxf_v, idxp_v,
             tot_v, tot_sh, prev0_v, rows0, rows1, rows2, outbA, outbB,
             gsem0, gsem1, gsem2, osemA, osemB, psem):
        rows = (rows0, rows1, rows2)
        gsem = (gsem0, gsem1, gsem2)
        outb = (outbA, outbB)
        osem = (osemA, osemB)
        sid = lax.axis_index("s")
        wid = sid * NC + lax.axis_index("c")
        batch = wid % B
        chunk = wid // B
        start = chunk * TCH
        obase = batch * T + start

        pltpu.sync_copy(b_hbm.at[pl.ds(batch * T, T)],
                        b_v.at[pl.ds(LANES, T)])
        pltpu.sync_copy(p_hbm.at[pl.ds(batch * T + start, TCH)],
                        p_v.at[pl.ds(0, TCH)])

        zbase = batch * L
        zeros = jnp.zeros((LANES,), jnp.int32)
        b_v[pl.ds(0, LANES)] = zeros

        @pl.when(chunk == 0)
        def _():
            lane = jnp.arange(LANES, dtype=jnp.int32)
            p0 = p_v[pl.ds(0, LANES)]
            p_v[pl.ds(0, LANES)] = jnp.where(lane == 0, 1.0, p0)

        c_v[pl.ds(0, LANES)] = zeros

        def shift_body(m, carry):
            c_v[pl.ds(LANES + m * LANES, LANES)] = b_v[pl.ds(
                LANES - 1 + start + m * LANES, LANES)]
            return carry

        lax.fori_loop(0, NVC, shift_body, jnp.int32(0))

        s = 1
        while s < TCH:
            lo = s // LANES

            def scan_body(jj, carry, s=s, lo=lo):
                j = NVC - 1 - jj
                o = LANES + j * LANES
                c_v[pl.ds(o, LANES)] = (c_v[pl.ds(o, LANES)] +
                                        c_v[pl.ds(o - s, LANES)])
                return carry

            lax.fori_loop(0, NVC - lo, scan_body, jnp.int32(0))
            s *= 2

        pltpu.sync_copy(c_v.at[pl.ds(LANES + TCH - LANES, LANES)],
                        tot_sh.at[sid])
        plsc.subcore_barrier()
        pltpu.sync_copy(tot_sh, tot_v)
        off = jnp.int32(0)
        for k in range(1, CPB):
            vk = tot_v[jnp.maximum(sid - 4 * k, 0), pl.ds(0, LANES)][15]
            off = off + jnp.where(chunk >= k, vk, 0)

        idxp_v[pl.ds(0, LANES)] = (
            jnp.minimum(c_v[pl.ds(LANES - 1, LANES)] + off, L - 1) + zbase)

        def idx_body(m, carry):
            o = m * LANES
            e = c_v[pl.ds(LANES + o, LANES)] + off
            idxf_v[pl.ds(LANES + start + o, LANES)] = (
                jnp.minimum(e, L - 1) + zbase)
            return carry

        lax.fori_loop(0, NVC, idx_body, jnp.int32(0))

        pltpu.async_copy(z_hbm.at[idxp_v.at[pl.ds(0, 8)]], prev0_v,
                         psem).wait()

        def g_src(i):
            return z_hbm.at[idxf_v.at[pl.ds(LANES + start + i * S, S)]]

        pltpu.async_copy(g_src(0), rows[0], gsem[0])
        pltpu.async_copy(g_src(1), rows[1], gsem[1])
        pltpu.async_copy(g_src(2), rows[2], gsem[2])

        def blend(s0, rbuf, obuf, prev):
            def rb(k, prev, rbuf=rbuf, obuf=obuf):
                pk = p_v[pl.ds(s0 + k, LANES)][0]
                qk = 1.0 - pk
                new = []
                for u in range(UD):
                    sl = pl.ds(u * LANES, LANES)
                    cu = rbuf[k, sl]
                    obuf[k, sl] = pk * cu + qk * prev[u]
                    new.append(cu)
                return tuple(new)

            return lax.fori_loop(0, S, rb, prev)

        prev = tuple(prev0_v[0, pl.ds(u * LANES, LANES)] for u in range(UD))

        for i in range(NSUB):
            ri, oi = i % 3, i % 2
            pltpu.make_async_copy(z_hbm.at[pl.ds(0, S)], rows[ri],
                                  gsem[ri]).wait()
            if i >= 2:
                pltpu.make_async_copy(outb[oi], out_hbm.at[pl.ds(0, S)],
                                      osem[oi]).wait()
            prev = blend(i * S, rows[ri], outb[oi], prev)
            pltpu.async_copy(outb[oi], out_hbm.at[pl.ds(obase + i * S, S)],
                             osem[oi])
            if i + 3 < NSUB:
                pltpu.async_copy(g_src(i + 3), rows[(i + 3) % 3],
                                 gsem[(i + 3) % 3])

        pltpu.make_async_copy(outbA, out_hbm.at[pl.ds(0, S)], osemA).wait()
        pltpu.make_async_copy(outbB, out_hbm.at[pl.ds(0, S)], osemB).wait()

    return body(zf, pf, bf)


def kernel(z, p, b, original_len):
    B, L, D = z.shape
    T = b.shape[1]
    zf = z.reshape(B * L, D)
    pf = p.reshape(B * T)
    bf = b.reshape(B * T)
    out = _dechunk_sc(zf, pf, bf, B, T, L, D)
    return out.reshape(B, T, D)


# speedup vs baseline: 1.1059x; 1.1059x over previous
import functools

import jax
import jax.numpy as jnp
from jax import lax
from jax.experimental import pallas as pl
from jax.experimental.pallas import tpu as pltpu
from jax.experimental.pallas import tpu_sc as plsc

LANES = 16


def _dechunk_sc(zf, pf, bf, B, T, L, D):
    info = plsc.get_sparse_core_info()
    NC, NS = info.num_cores, info.num_subcores
    NW = NC * NS
    CPB = NW // B
    TCH = T // CPB
    S = 32
    NSUB = TCH // S
    NVC = TCH // LANES
    UD = D // LANES

    mesh = plsc.VectorSubcoreMesh(core_axis_name="c", subcore_axis_name="s")

    @functools.partial(
        pl.kernel,
        out_type=jax.ShapeDtypeStruct((B * T, D), jnp.float32),
        mesh=mesh,
        scratch_types=[
            pltpu.VMEM((T + LANES,), jnp.int32),
            pltpu.VMEM((TCH + LANES,), jnp.float32),
            pltpu.VMEM((TCH + 2 * LANES,), jnp.int32),
            pltpu.VMEM((T + 2 * LANES,), jnp.int32),
            pltpu.VMEM((LANES,), jnp.int32),
            pltpu.VMEM((16, LANES), jnp.int32),
            pltpu.VMEM_SHARED((16, LANES), jnp.int32),
            pltpu.VMEM((8, D), jnp.float32),
            pltpu.VMEM((S, D), jnp.float32),
            pltpu.VMEM((S, D), jnp.float32),
            pltpu.VMEM((S, D), jnp.float32),
            pltpu.VMEM((S, D), jnp.float32),
            pltpu.VMEM((S, D), jnp.float32),
            pltpu.VMEM((S, D), jnp.float32),
            pltpu.SemaphoreType.DMA,
            pltpu.SemaphoreType.DMA,
            pltpu.SemaphoreType.DMA,
            pltpu.SemaphoreType.DMA,
            pltpu.SemaphoreType.DMA,
            pltpu.SemaphoreType.DMA,
            pltpu.SemaphoreType.DMA,
        ],
    )
    def body(z_hbm, p_hbm, b_hbm, out_hbm, b_v, p_v, c_v, idxf_v, idxp_v,
             tot_v, tot_sh, prev0_v, rows0, rows1, rows2, rows3,
             outbA, outbB, gsem0, gsem1, gsem2, gsem3, osemA, osemB, psem):
        rows = (rows0, rows1, rows2, rows3)
        gsem = (gsem0, gsem1, gsem2, gsem3)
        outb = (outbA, outbB)
        osem = (osemA, osemB)

        sid = lax.axis_index("s")
        wid = sid * NC + lax.axis_index("c")
        batch = wid % B
        chunk = wid // B
        start = chunk * TCH
        obase = batch * T + start

        pltpu.sync_copy(b_hbm.at[pl.ds(batch * T, T)],
                        b_v.at[pl.ds(LANES, T)])
        pltpu.sync_copy(p_hbm.at[pl.ds(batch * T + start, TCH)],
                        p_v.at[pl.ds(0, TCH)])

        zbase = batch * L
        zeros = jnp.zeros((LANES,), jnp.int32)
        b_v[pl.ds(0, LANES)] = zeros

        @pl.when(chunk == 0)
        def _():
            lane = jnp.arange(LANES, dtype=jnp.int32)
            p0 = p_v[pl.ds(0, LANES)]
            p_v[pl.ds(0, LANES)] = jnp.where(lane == 0, 1.0, p0)

        c_v[pl.ds(0, LANES)] = zeros

        def shift_body(m, carry):
            c_v[pl.ds(LANES + m * LANES, LANES)] = b_v[pl.ds(
                LANES - 1 + start + m * LANES, LANES)]
            return carry

        lax.fori_loop(0, NVC, shift_body, jnp.int32(0))

        s = 1
        while s < TCH:
            lo = s // LANES

            def scan_body(jj, carry, s=s, lo=lo):
                j = NVC - 1 - jj
                o = LANES + j * LANES
                c_v[pl.ds(o, LANES)] = (c_v[pl.ds(o, LANES)] +
                                        c_v[pl.ds(o - s, LANES)])
                return carry

            lax.fori_loop(0, NVC - lo, scan_body, jnp.int32(0))
            s *= 2

        pltpu.sync_copy(c_v.at[pl.ds(LANES + TCH - LANES, LANES)],
                        tot_sh.at[sid])
        plsc.subcore_barrier()
        pltpu.sync_copy(tot_sh, tot_v)
        off = jnp.int32(0)
        for k in range(1, CPB):
            vk = tot_v[jnp.maximum(sid - 4 * k, 0), pl.ds(0, LANES)][15]
            off = off + jnp.where(chunk >= k, vk, 0)

        idxp_v[pl.ds(0, LANES)] = (
            jnp.minimum(c_v[pl.ds(LANES - 1, LANES)] + off, L - 1) + zbase)

        def idx_body(m, carry):
            o = m * LANES
            e = c_v[pl.ds(LANES + o, LANES)] + off
            idxf_v[pl.ds(LANES + start + o, LANES)] = (
                jnp.minimum(e, L - 1) + zbase)
            return carry

        lax.fori_loop(0, NVC, idx_body, jnp.int32(0))

        pltpu.async_copy(z_hbm.at[idxp_v.at[pl.ds(0, 8)]], prev0_v,
                         psem).wait()

        def g_src(i):
            return z_hbm.at[idxf_v.at[pl.ds(LANES + start + i * S, S)]]

        for j in range(4):
            pltpu.async_copy(g_src(j), rows[j], gsem[j])

        def blend(s0, rbuf, obuf, prev):
            def rb(k, prev, rbuf=rbuf, obuf=obuf):
                pk = p_v[pl.ds(s0 + k, LANES)][0]
                qk = 1.0 - pk
                new = []
                for u in range(UD):
                    sl = pl.ds(u * LANES, LANES)
                    cu = rbuf[k, sl]
                    obuf[k, sl] = pk * cu + qk * prev[u]
                    new.append(cu)
                return tuple(new)

            return lax.fori_loop(0, S, rb, prev)

        def quarter(ii, j, prev):
            i = 4 * ii + j
            rbuf, gs = rows[j], gsem[j]
            obuf, os_ = outb[j % 2], osem[j % 2]
            pltpu.make_async_copy(z_hbm.at[pl.ds(0, S)], rbuf, gs).wait()

            if j >= 2:
                pltpu.make_async_copy(obuf, out_hbm.at[pl.ds(0, S)],
                                      os_).wait()
            else:

                @pl.when(ii > 0)
                def _():
                    pltpu.make_async_copy(obuf, out_hbm.at[pl.ds(0, S)],
                                          os_).wait()

            prev = blend(i * S, rbuf, obuf, prev)
            pltpu.async_copy(obuf, out_hbm.at[pl.ds(obase + i * S, S)], os_)

            @pl.when(i + 4 < NSUB)
            def _():
                pltpu.async_copy(g_src(i + 4), rbuf, gs)

            return prev

        prev0 = tuple(prev0_v[0, pl.ds(u * LANES, LANES)] for u in range(UD))

        def quad_body(ii, prev):
            for j in range(4):
                prev = quarter(ii, j, prev)
            return prev

        lax.fori_loop(0, NSUB // 4, quad_body, prev0)

        pltpu.make_async_copy(outbA, out_hbm.at[pl.ds(0, S)], osemA).wait()
        pltpu.make_async_copy(outbB, out_hbm.at[pl.ds(0, S)], osemB).wait()

    return body(zf, pf, bf)


def kernel(z, p, b, original_len):
    B, L, D = z.shape
    T = b.shape[1]
    zf = z.reshape(B * L, D)
    pf = p.reshape(B * T)
    bf = b.reshape(B * T)
    out = _dechunk_sc(zf, pf, bf, B, T, L, D)
    return out.reshape(B, T, D)


# speedup vs baseline: 1.1202x; 1.0129x over previous
import functools

import jax
import jax.numpy as jnp
from jax import lax
from jax.experimental import pallas as pl
from jax.experimental.pallas import tpu as pltpu
from jax.experimental.pallas import tpu_sc as plsc

LANES = 16


def _dechunk_sc(zf, pf, bf, B, T, L, D):
    info = plsc.get_sparse_core_info()
    NC, NS = info.num_cores, info.num_subcores
    NW = NC * NS
    CPB = NW // B
    TCH = T // CPB
    S = 32
    NSUB = TCH // S
    NVC = TCH // LANES
    UD = D // LANES

    mesh = plsc.VectorSubcoreMesh(core_axis_name="c", subcore_axis_name="s")

    @functools.partial(
        pl.kernel,
        out_type=jax.ShapeDtypeStruct((B * T, D), jnp.float32),
        mesh=mesh,
        scratch_types=[
            pltpu.VMEM((T + LANES,), jnp.int32),
            pltpu.VMEM((TCH + LANES,), jnp.float32),
            pltpu.VMEM((TCH + 2 * LANES,), jnp.int32),
            pltpu.VMEM((T + 2 * LANES,), jnp.int32),
            pltpu.VMEM((LANES,), jnp.int32),
            pltpu.VMEM((16, LANES), jnp.int32),
            pltpu.VMEM_SHARED((16, LANES), jnp.int32),
            pltpu.VMEM((8, D), jnp.float32),
            pltpu.VMEM((S, D), jnp.float32),
            pltpu.VMEM((S, D), jnp.float32),
            pltpu.VMEM((S, D), jnp.float32),
            pltpu.VMEM((S, D), jnp.float32),
            pltpu.VMEM((S, D), jnp.float32),
            pltpu.VMEM((S, D), jnp.float32),
            pltpu.SemaphoreType.DMA,
            pltpu.SemaphoreType.DMA,
            pltpu.SemaphoreType.DMA,
            pltpu.SemaphoreType.DMA,
            pltpu.SemaphoreType.DMA,
            pltpu.SemaphoreType.DMA,
            pltpu.SemaphoreType.DMA,
        ],
    )
    def body(z_hbm, p_hbm, b_hbm, out_hbm, b_v, p_v, c_v, idxf_v, idxp_v,
             tot_v, tot_sh, prev0_v, rows0, rows1, rows2, rows3,
             outbA, outbB, gsem0, gsem1, gsem2, gsem3, osemA, osemB, psem):
        rows = (rows0, rows1, rows2, rows3)
        gsem = (gsem0, gsem1, gsem2, gsem3)
        outb = (outbA, outbB)
        osem = (osemA, osemB)

        sid = lax.axis_index("s")
        wid = sid * NC + lax.axis_index("c")
        batch = wid % B
        chunk = wid // B
        start = chunk * TCH
        obase = batch * T + start

        pltpu.sync_copy(b_hbm.at[pl.ds(batch * T, T)],
                        b_v.at[pl.ds(LANES, T)])
        pltpu.sync_copy(p_hbm.at[pl.ds(batch * T + start, TCH)],
                        p_v.at[pl.ds(0, TCH)])

        zbase = batch * L
        zeros = jnp.zeros((LANES,), jnp.int32)
        b_v[pl.ds(0, LANES)] = zeros

        @pl.when(chunk == 0)
        def _():
            lane = jnp.arange(LANES, dtype=jnp.int32)
            p0 = p_v[pl.ds(0, LANES)]
            p_v[pl.ds(0, LANES)] = jnp.where(lane == 0, 1.0, p0)

        c_v[pl.ds(0, LANES)] = zeros

        def shift_body(m, carry):
            c_v[pl.ds(LANES + m * LANES, LANES)] = b_v[pl.ds(
                LANES - 1 + start + m * LANES, LANES)]
            return carry

        lax.fori_loop(0, NVC, shift_body, jnp.int32(0))

        s = 1
        while s < TCH:
            lo = s // LANES

            def scan_body(jj, carry, s=s, lo=lo):
                j = NVC - 1 - jj
                o = LANES + j * LANES
                c_v[pl.ds(o, LANES)] = (c_v[pl.ds(o, LANES)] +
                                        c_v[pl.ds(o - s, LANES)])
                return carry

            lax.fori_loop(0, NVC - lo, scan_body, jnp.int32(0))
            s *= 2

        pltpu.sync_copy(c_v.at[pl.ds(LANES + TCH - LANES, LANES)],
                        tot_sh.at[sid])
        plsc.subcore_barrier()
        pltpu.sync_copy(tot_sh, tot_v)
        off = jnp.int32(0)
        for k in range(1, CPB):
            vk = tot_v[jnp.maximum(sid - 4 * k, 0), pl.ds(0, LANES)][15]
            off = off + jnp.where(chunk >= k, vk, 0)

        idxp_v[pl.ds(0, LANES)] = (
            jnp.minimum(c_v[pl.ds(LANES - 1, LANES)] + off, L - 1) + zbase)

        def idx_body(m, carry):
            o = m * LANES
            e = c_v[pl.ds(LANES + o, LANES)] + off
            idxf_v[pl.ds(LANES + start + o, LANES)] = (
                jnp.minimum(e, L - 1) + zbase)
            return carry

        lax.fori_loop(0, NVC, idx_body, jnp.int32(0))

        prev0_cp = pltpu.async_copy(z_hbm.at[idxp_v.at[pl.ds(0, 8)]],
                                    prev0_v, psem)

        def g_src(i):
            return z_hbm.at[idxf_v.at[pl.ds(LANES + start + i * S, S)]]

        for j in range(4):
            pltpu.async_copy(g_src(j), rows[j], gsem[j])
        prev0_cp.wait()

        def blend(s0, rbuf, obuf, prev):
            def rb(k, prev, rbuf=rbuf, obuf=obuf):
                pk = p_v[pl.ds(s0 + k, LANES)][0]
                qk = 1.0 - pk
                new = []
                for u in range(UD):
                    sl = pl.ds(u * LANES, LANES)
                    cu = rbuf[k, sl]
                    obuf[k, sl] = pk * cu + qk * prev[u]
                    new.append(cu)
                return tuple(new)

            return lax.fori_loop(0, S, rb, prev)

        def quarter(ii, j, prev):
            i = 4 * ii + j
            rbuf, gs = rows[j], gsem[j]
            obuf, os_ = outb[j % 2], osem[j % 2]
            pltpu.make_async_copy(z_hbm.at[pl.ds(0, S)], rbuf, gs).wait()

            if j >= 2:
                pltpu.make_async_copy(obuf, out_hbm.at[pl.ds(0, S)],
                                      os_).wait()
            else:

                @pl.when(ii > 0)
                def _():
                    pltpu.make_async_copy(obuf, out_hbm.at[pl.ds(0, S)],
                                          os_).wait()

            prev = blend(i * S, rbuf, obuf, prev)
            pltpu.async_copy(obuf, out_hbm.at[pl.ds(obase + i * S, S)], os_)

            @pl.when(i + 4 < NSUB)
            def _():
                pltpu.async_copy(g_src(i + 4), rbuf, gs)

            return prev

        prev0 = tuple(prev0_v[0, pl.ds(u * LANES, LANES)] for u in range(UD))

        def quad_body(ii, prev):
            for j in range(4):
                prev = quarter(ii, j, prev)
            return prev

        lax.fori_loop(0, NSUB // 4, quad_body, prev0)

        pltpu.make_async_copy(outbA, out_hbm.at[pl.ds(0, S)], osemA).wait()
        pltpu.make_async_copy(outbB, out_hbm.at[pl.ds(0, S)], osemB).wait()

    return body(zf, pf, bf)


def kernel(z, p, b, original_len):
    B, L, D = z.shape
    T = b.shape[1]
    zf = z.reshape(B * L, D)
    pf = p.reshape(B * T)
    bf = b.reshape(B * T)
    out = _dechunk_sc(zf, pf, bf, B, T, L, D)
    return out.reshape(B, T, D)


# speedup vs baseline: 1.1301x; 1.0088x over previous
import functools

import jax
import jax.numpy as jnp
from jax import lax
from jax.experimental import pallas as pl
from jax.experimental.pallas import tpu as pltpu
from jax.experimental.pallas import tpu_sc as plsc

LANES = 16


def _dechunk_sc(zf, pf, bf, B, T, L, D):
    info = plsc.get_sparse_core_info()
    NC, NS = info.num_cores, info.num_subcores
    NW = NC * NS
    CPB = NW // B
    TCH = T // CPB
    S = 32
    NSUB = TCH // S
    NVC = TCH // LANES
    UD = D // LANES

    mesh = plsc.VectorSubcoreMesh(core_axis_name="c", subcore_axis_name="s")

    @functools.partial(
        pl.kernel,
        out_type=jax.ShapeDtypeStruct((B * T, D), jnp.float32),
        mesh=mesh,
        scratch_types=[
            pltpu.VMEM((T + LANES,), jnp.int32),
            pltpu.VMEM((TCH + LANES,), jnp.float32),
            pltpu.VMEM((TCH + 2 * LANES,), jnp.int32),
            pltpu.VMEM((T + 2 * LANES,), jnp.int32),
            pltpu.VMEM((LANES,), jnp.int32),
            pltpu.VMEM((16, LANES), jnp.int32),
            pltpu.VMEM_SHARED((16, LANES), jnp.int32),
            pltpu.VMEM((8, D), jnp.float32),
            pltpu.VMEM((S, D), jnp.float32),
            pltpu.VMEM((S, D), jnp.float32),
            pltpu.VMEM((S, D), jnp.float32),
            pltpu.VMEM((S, D), jnp.float32),
            pltpu.VMEM((S, D), jnp.float32),
            pltpu.VMEM((S, D), jnp.float32),
            pltpu.SemaphoreType.DMA,
            pltpu.SemaphoreType.DMA,
            pltpu.SemaphoreType.DMA,
            pltpu.SemaphoreType.DMA,
            pltpu.SemaphoreType.DMA,
            pltpu.SemaphoreType.DMA,
            pltpu.SemaphoreType.DMA,
            pltpu.SemaphoreType.DMA,
        ],
    )
    def body(z_hbm, p_hbm, b_hbm, out_hbm, b_v, p_v, c_v, idxf_v, idxp_v,
             tot_v, tot_sh, prev0_v, rows0, rows1, rows2, rows3,
             outbA, outbB, gsem0, gsem1, gsem2, gsem3, osemA, osemB, psem,
             plsem):
        rows = (rows0, rows1, rows2, rows3)
        gsem = (gsem0, gsem1, gsem2, gsem3)
        outb = (outbA, outbB)
        osem = (osemA, osemB)

        sid = lax.axis_index("s")
        wid = sid * NC + lax.axis_index("c")
        batch = wid % B
        chunk = wid // B
        start = chunk * TCH
        obase = batch * T + start

        p_cp = pltpu.async_copy(p_hbm.at[pl.ds(batch * T + start, TCH)],
                                p_v.at[pl.ds(0, TCH)], plsem)
        pltpu.sync_copy(b_hbm.at[pl.ds(batch * T, T)],
                        b_v.at[pl.ds(LANES, T)])

        zbase = batch * L
        zeros = jnp.zeros((LANES,), jnp.int32)
        b_v[pl.ds(0, LANES)] = zeros

        c_v[pl.ds(0, LANES)] = zeros

        def shift_body(m, carry):
            c_v[pl.ds(LANES + m * LANES, LANES)] = b_v[pl.ds(
                LANES - 1 + start + m * LANES, LANES)]
            return carry

        lax.fori_loop(0, NVC, shift_body, jnp.int32(0))

        s = 1
        while s < TCH:
            lo = s // LANES

            def scan_body(jj, carry, s=s, lo=lo):
                j = NVC - 1 - jj
                o = LANES + j * LANES
                c_v[pl.ds(o, LANES)] = (c_v[pl.ds(o, LANES)] +
                                        c_v[pl.ds(o - s, LANES)])
                return carry

            lax.fori_loop(0, NVC - lo, scan_body, jnp.int32(0))
            s *= 2

        pltpu.sync_copy(c_v.at[pl.ds(LANES + TCH - LANES, LANES)],
                        tot_sh.at[sid])
        plsc.subcore_barrier()
        pltpu.sync_copy(tot_sh, tot_v)
        off = jnp.int32(0)
        for k in range(1, CPB):
            vk = tot_v[jnp.maximum(sid - 4 * k, 0), pl.ds(0, LANES)][15]
            off = off + jnp.where(chunk >= k, vk, 0)

        idxp_v[pl.ds(0, LANES)] = (
            jnp.minimum(c_v[pl.ds(LANES - 1, LANES)] + off, L - 1) + zbase)

        def idx_body(m, carry):
            o = m * LANES
            e = c_v[pl.ds(LANES + o, LANES)] + off
            idxf_v[pl.ds(LANES + start + o, LANES)] = (
                jnp.minimum(e, L - 1) + zbase)
            return carry

        lax.fori_loop(0, NVC, idx_body, jnp.int32(0))

        prev0_cp = pltpu.async_copy(z_hbm.at[idxp_v.at[pl.ds(0, 8)]],
                                    prev0_v, psem)

        def g_src(i):
            return z_hbm.at[idxf_v.at[pl.ds(LANES + start + i * S, S)]]

        for j in range(4):
            pltpu.async_copy(g_src(j), rows[j], gsem[j])
        prev0_cp.wait()
        p_cp.wait()

        @pl.when(chunk == 0)
        def _():
            lane = jnp.arange(LANES, dtype=jnp.int32)
            p0 = p_v[pl.ds(0, LANES)]
            p_v[pl.ds(0, LANES)] = jnp.where(lane == 0, 1.0, p0)

        def blend(s0, rbuf, obuf, prev):
            def rb(k, prev, rbuf=rbuf, obuf=obuf):
                pk = p_v[pl.ds(s0 + k, LANES)][0]
                qk = 1.0 - pk
                new = []
                for u in range(UD):
                    sl = pl.ds(u * LANES, LANES)
                    cu = rbuf[k, sl]
                    obuf[k, sl] = pk * cu + qk * prev[u]
                    new.append(cu)
                return tuple(new)

            return lax.fori_loop(0, S, rb, prev)

        def quarter(ii, j, prev):
            i = 4 * ii + j
            rbuf, gs = rows[j], gsem[j]
            obuf, os_ = outb[j % 2], osem[j % 2]
            pltpu.make_async_copy(z_hbm.at[pl.ds(0, S)], rbuf, gs).wait()

            if j >= 2:
                pltpu.make_async_copy(obuf, out_hbm.at[pl.ds(0, S)],
                                      os_).wait()
            else:

                @pl.when(ii > 0)
                def _():
                    pltpu.make_async_copy(obuf, out_hbm.at[pl.ds(0, S)],
                                          os_).wait()

            prev = blend(i * S, rbuf, obuf, prev)
            pltpu.async_copy(obuf, out_hbm.at[pl.ds(obase + i * S, S)], os_)

            @pl.when(i + 4 < NSUB)
            def _():
                pltpu.async_copy(g_src(i + 4), rbuf, gs)

            return prev

        prev0 = tuple(prev0_v[0, pl.ds(u * LANES, LANES)] for u in range(UD))

        def quad_body(ii, prev):
            for j in range(4):
                prev = quarter(ii, j, prev)
            return prev

        lax.fori_loop(0, NSUB // 4, quad_body, prev0)

        pltpu.make_async_copy(outbA, out_hbm.at[pl.ds(0, S)], osemA).wait()
        pltpu.make_async_copy(outbB, out_hbm.at[pl.ds(0, S)], osemB).wait()

    return body(zf, pf, bf)


def kernel(z, p, b, original_len):
    B, L, D = z.shape
    T = b.shape[1]
    zf = z.reshape(B * L, D)
    pf = p.reshape(B * T)
    bf = b.reshape(B * T)
    out = _dechunk_sc(zf, pf, bf, B, T, L, D)
    return out.reshape(B, T, D)
